# Initial kernel scaffold; baseline (speedup 1.0000x reference)
import jax
import jax.numpy as jnp
from jax.experimental import pallas as pl


def kernel(x, x_org, edge_index, W_in, b_in, W_g0, b_g0, ln_g0, ln_b0, W_g1, b_g1, ln_g1, ln_b1, W_g2, b_g2, ln_g2, ln_b2, W_lin, b_lin, W_res):
    raise NotImplementedError("write your pallas kernel here")


# speedup vs baseline: 7.0416x; 7.0416x over previous
import functools

import jax
import jax.numpy as jnp
from jax import lax
from jax.experimental import pallas as pl
from jax.experimental.pallas import tpu as pltpu
from jax.experimental.pallas import tpu_sc as plsc

N = 10000
D = 128
NC = 2
NS = 16
NW = NC * NS
CH = 128
NP = 10240
DUMMY_DST = N + 64
DEGW = 16


def _pad_edges(src, dst, e_pad):
    src_p = jnp.concatenate([src, jnp.zeros((e_pad,), jnp.int32)])
    dst_p = jnp.concatenate([dst, jnp.full((e_pad,), DUMMY_DST, jnp.int32)])
    return src_p, dst_p


def _sc_scatter_body(nch, table, src_hbm, dst_hbm, zeros_hbm,
                     pout, src_v, dst_v, rows_v, acc_sh, sem):
    c = lax.axis_index("c")
    s = lax.axis_index("s")

    pltpu.sync_copy(zeros_hbm, rows_v)
    stripe = NP // NS
    base_r = s * stripe
    for k in range(stripe // CH):
        pltpu.sync_copy(rows_v, acc_sh.at[pl.ds(base_r + k * CH, CH)])
    plsc.subcore_barrier()

    e_pw = nch * CH
    base_e = (c * NS + s) * e_pw

    def _chunk(i, _):
        off = base_e + i * CH
        pltpu.sync_copy(src_hbm.at[pl.ds(off, CH)], src_v)
        pltpu.sync_copy(dst_hbm.at[pl.ds(off, CH)], dst_v)
        pltpu.async_copy(table.at[src_v], rows_v, sem).wait()
        pltpu.sync_copy(rows_v, acc_sh.at[dst_v], add=True)
        return 0
    lax.fori_loop(0, nch, _chunk, 0)
    plsc.subcore_barrier()

    for k in range(stripe // CH):
        r0 = base_r + k * CH
        pltpu.sync_copy(acc_sh.at[pl.ds(r0, CH)], rows_v)
        pltpu.sync_copy(rows_v, pout.at[c, pl.ds(r0, CH)])


def _make_sc_scatter(nch):
    mesh = plsc.VectorSubcoreMesh(core_axis_name="c", subcore_axis_name="s")
    return pl.kernel(
        functools.partial(_sc_scatter_body, nch),
        out_type=[jax.ShapeDtypeStruct((NC, NP, D), jnp.float32)],
        mesh=mesh,
        scratch_types=[pltpu.VMEM((CH,), jnp.int32),
                       pltpu.VMEM((CH,), jnp.int32),
                       pltpu.VMEM((CH, D), jnp.float32),
                       pltpu.VMEM_SHARED((NP, D), jnp.float32),
                       pltpu.SemaphoreType.DMA])


def _sc_deg_body(nch, dst_hbm, zeros_hbm, ones_hbm,
                 dout, dst_v, ones_v, acc_sh):
    c = lax.axis_index("c")
    s = lax.axis_index("s")

    pltpu.sync_copy(zeros_hbm, ones_v)
    stripe = NP // NS
    base_r = s * stripe
    for k in range(stripe // CH):
        pltpu.sync_copy(ones_v, acc_sh.at[pl.ds(base_r + k * CH, CH)])
    pltpu.sync_copy(ones_hbm, ones_v)
    plsc.subcore_barrier()

    e_pw = nch * CH
    base_e = (c * NS + s) * e_pw

    def _chunk(i, _):
        off = base_e + i * CH
        pltpu.sync_copy(dst_hbm.at[pl.ds(off, CH)], dst_v)
        pltpu.sync_copy(ones_v, acc_sh.at[dst_v], add=True)
        return 0
    lax.fori_loop(0, nch, _chunk, 0)
    plsc.subcore_barrier()

    for k in range(stripe // CH):
        r0 = base_r + k * CH
        pltpu.sync_copy(acc_sh.at[pl.ds(r0, CH)], ones_v)
        pltpu.sync_copy(ones_v, dout.at[c, pl.ds(r0, CH)])


def _make_sc_deg(nch):
    mesh = plsc.VectorSubcoreMesh(core_axis_name="c", subcore_axis_name="s")
    return pl.kernel(
        functools.partial(_sc_deg_body, nch),
        out_type=[jax.ShapeDtypeStruct((NC, NP, D), jnp.float32)],
        mesh=mesh,
        scratch_types=[pltpu.VMEM((CH,), jnp.int32),
                       pltpu.VMEM((CH, D), jnp.float32),
                       pltpu.VMEM_SHARED((NP, D), jnp.float32)])


BR = 400


def _tc1_body(x, xorg, wres, win, bin_, wg0, xr, u1):
    xr[...] = jnp.dot(xorg[...], wres[...], preferred_element_type=jnp.float32)
    h0 = jnp.maximum(
        jnp.dot(x[...], win[...], preferred_element_type=jnp.float32)
        + bin_[...], 0.0)
    u1[...] = jnp.dot(h0, wg0[...], preferred_element_type=jnp.float32)


def _tc2_body(rp0, rp1, dg0, dg1, u1, res, t1, dinv):
    res[...] = rp0[0] + rp1[0]
    deg = dg0[0][:, 0:1] + dg1[0][:, 0:1] + 1.0
    dv = lax.rsqrt(deg)
    dinv[...] = jnp.broadcast_to(dv, (BR, D))
    t1[...] = dv * u1[...]


def _layer_math(sp0, sp1, t, dinv, b, g, be):
    a = dinv[...] * (sp0[0] + sp1[0] + t[...]) + b[...]
    mu = jnp.mean(a, axis=-1, keepdims=True)
    var = jnp.mean((a - mu) ** 2, axis=-1, keepdims=True)
    xhat = (a - mu) * lax.rsqrt(var + 1e-5) * g[...] + be[...]
    return jnp.maximum(xhat, 0.0)


def _tc_mid_body(sp0, sp1, t, dinv, b, g, be, wn, tn):
    h = _layer_math(sp0, sp1, t, dinv, b, g, be)
    tn[...] = dinv[...] * jnp.dot(h, wn[...],
                                  preferred_element_type=jnp.float32)


def _tc_fin_body(sp0, sp1, t, dinv, b, g, be, wlin, blin, out):
    h = _layer_math(sp0, sp1, t, dinv, b, g, be)
    out[...] = (jnp.dot(h, wlin[...], preferred_element_type=jnp.float32)
                + blin[...])


def _row_spec():
    return pl.BlockSpec((BR, D), lambda i: (i, 0))


def _part_spec(core):
    return pl.BlockSpec((1, BR, D), lambda i, c=core: (c, i, 0))


def _full_spec(shape):
    return pl.BlockSpec(shape, lambda i: tuple(0 for _ in shape))


def _vec_spec():
    return pl.BlockSpec((1, D), lambda i: (0, 0))


def kernel(x, x_org, edge_index, W_in, b_in, W_g0, b_g0, ln_g0, ln_b0,
           W_g1, b_g1, ln_g1, ln_b1, W_g2, b_g2, ln_g2, ln_b2,
           W_lin, b_lin, W_res):
    n = x.shape[0]
    e = edge_index.shape[1]
    src, dst = edge_index[0], edge_index[1]
    e_pw_pad = -(-e // (NW * CH)) * CH
    nch = e_pw_pad // CH
    ep = e_pw_pad * NW
    src_p, dst_p = _pad_edges(src, dst, ep - e)

    grid = n // BR
    b_in2 = b_in.reshape(1, D)
    b_g02 = b_g0.reshape(1, D)
    b_g12 = b_g1.reshape(1, D)
    b_g22 = b_g2.reshape(1, D)
    g0 = ln_g0.reshape(1, D)
    be0 = ln_b0.reshape(1, D)
    g1 = ln_g1.reshape(1, D)
    be1 = ln_b1.reshape(1, D)
    g2 = ln_g2.reshape(1, D)
    be2 = ln_b2.reshape(1, D)
    b_lin2 = b_lin.reshape(1, D)

    xr, u1 = pl.pallas_call(
        _tc1_body,
        grid=(grid,),
        in_specs=[_row_spec(), _row_spec(), _full_spec((D, D)),
                  _full_spec((D, D)), _vec_spec(), _full_spec((D, D))],
        out_specs=[_row_spec(), _row_spec()],
        out_shape=[jax.ShapeDtypeStruct((n, D), jnp.float32),
                   jax.ShapeDtypeStruct((n, D), jnp.float32)],
    )(x, x_org, W_res, W_in, b_in2, W_g0)

    zeros_blk = jnp.zeros((CH, D), jnp.float32)
    ones_blk = jnp.ones((CH, D), jnp.float32)

    sc_gs = _make_sc_scatter(nch)

    dg, = _make_sc_deg(nch)(dst_p, zeros_blk, ones_blk)
    rp, = sc_gs(xr, src_p, dst_p, zeros_blk)

    res, t1, dinv = pl.pallas_call(
        _tc2_body,
        grid=(grid,),
        in_specs=[_part_spec(0), _part_spec(1),
                  _part_spec(0), _part_spec(1), _row_spec()],
        out_specs=[_row_spec(), _row_spec(), _row_spec()],
        out_shape=[jax.ShapeDtypeStruct((n, D), jnp.float32),
                   jax.ShapeDtypeStruct((n, D), jnp.float32),
                   jax.ShapeDtypeStruct((n, D), jnp.float32)],
    )(rp, rp, dg, dg, u1)

    def mid_layer(t, b2, g, be, wn):
        sp, = sc_gs(t, src_p, dst_p, zeros_blk)
        return pl.pallas_call(
            _tc_mid_body,
            grid=(grid,),
            in_specs=[_part_spec(0), _part_spec(1), _row_spec(), _row_spec(),
                      _vec_spec(), _vec_spec(), _vec_spec(),
                      _full_spec((D, D))],
            out_specs=[_row_spec()],
            out_shape=[jax.ShapeDtypeStruct((n, D), jnp.float32)],
        )(sp, sp, t, dinv, b2, g, be, wn)[0]

    t2 = mid_layer(t1, b_g02, g0, be0, W_g1)
    t3 = mid_layer(t2, b_g12, g1, be1, W_g2)

    sp, = sc_gs(t3, src_p, dst_p, zeros_blk)
    out = pl.pallas_call(
        _tc_fin_body,
        grid=(grid,),
        in_specs=[_part_spec(0), _part_spec(1), _row_spec(), _row_spec(),
                  _vec_spec(), _vec_spec(), _vec_spec(),
                  _full_spec((D, D)), _vec_spec()],
        out_specs=[_row_spec()],
        out_shape=[jax.ShapeDtypeStruct((n, D), jnp.float32)],
    )(sp, sp, t3, dinv, b_g22, g2, be2, W_lin, b_lin2)[0]

    return (out, res)
